# baseline (speedup 1.0000x reference)
import functools

import jax
import jax.numpy as jnp
from jax import lax
from jax.experimental import pallas as pl
from jax.experimental.pallas import tpu as pltpu
from jax.experimental.pallas import tpu_sc as plsc

_CHUNK = 512


@functools.lru_cache(maxsize=None)
def _make_gather(V, W, B):
    info = plsc.get_sparse_core_info()
    NC, NS = info.num_cores, info.num_subcores
    NW = NC * NS
    assert B % NW == 0
    b_per_w = B // NW
    C = _CHUNK
    assert b_per_w % (2 * C) == 0
    n_chunks = b_per_w // C
    n_pairs = n_chunks // 2
    mesh = plsc.VectorSubcoreMesh(core_axis_name="c", subcore_axis_name="s")

    @functools.partial(
        pl.kernel,
        mesh=mesh,
        compiler_params=pltpu.CompilerParams(use_tc_tiling_on_sc=False),
        out_type=jax.ShapeDtypeStruct((B, W), jnp.int32),
        scratch_types=[
            pltpu.VMEM((b_per_w,), jnp.int32),
            pltpu.VMEM((C, W), jnp.int32),
            pltpu.VMEM((C, W), jnp.int32),
            pltpu.SemaphoreType.DMA,
            pltpu.SemaphoreType.DMA,
            pltpu.SemaphoreType.DMA,
            pltpu.SemaphoreType.DMA,
        ],
    )
    def k(table_hbm, idx_hbm, out_hbm, idx_v, rows0, rows1, g0, g1, s0, s1):
        wid = lax.axis_index("s") * NC + lax.axis_index("c")
        base = wid * b_per_w
        rows = (rows0, rows1)
        gsem = (g0, g1)
        ssem = (s0, s1)

        pltpu.sync_copy(idx_hbm.at[pl.ds(base, b_per_w)], idx_v)

        def gather(g, b):
            return pltpu.make_async_copy(
                table_hbm.at[idx_v.at[pl.ds(g * C, C)]], rows[b], gsem[b]
            )

        def store(g, b):
            return pltpu.make_async_copy(
                rows[b], out_hbm.at[pl.ds(base + g * C, C)], ssem[b]
            )

        gather(0, 0).start()
        gather(1, 1).start()

        def body(p, carry):
            g = p * 2
            for b in (0, 1):
                gather(g + b, b).wait()
                store(g + b, b).start()
                store(g + b, b).wait()
                gather(g + b + 2, b).start()
            return carry

        lax.fori_loop(0, n_pairs - 1, body, 0)

        g = n_chunks - 2
        for b in (0, 1):
            gather(g + b, b).wait()
            store(g + b, b).start()
            store(g + b, b).wait()

    return k


def kernel(sequence, weight):
    Bs, H = sequence.shape
    V, D = weight.shape
    W = D // 2
    idx = sequence.reshape(-1).astype(jnp.int32)
    wpacked = jax.lax.bitcast_convert_type(
        weight.astype(jnp.bfloat16).reshape(V, W, 2), jnp.int32
    )
    out = _make_gather(V, W, Bs * H)(wpacked, idx)
    outf = jax.lax.bitcast_convert_type(out, jnp.bfloat16).reshape(
        Bs * H, D
    ).astype(jnp.float32)
    return outf.reshape(Bs, H, D)


# speedup vs baseline: 1.9557x; 1.9557x over previous
import functools

import jax
import jax.numpy as jnp
from jax import lax
from jax.experimental import pallas as pl
from jax.experimental.pallas import tpu as pltpu
from jax.experimental.pallas import tpu_sc as plsc

_CHUNK = 512


@functools.lru_cache(maxsize=None)
def _make_gather(V, D, B):
    info = plsc.get_sparse_core_info()
    NC, NS = info.num_cores, info.num_subcores
    NW = NC * NS
    assert B % NW == 0
    b_per_w = B // NW
    C = _CHUNK
    assert b_per_w % (2 * C) == 0
    n_chunks = b_per_w // C
    n_pairs = n_chunks // 2
    mesh = plsc.VectorSubcoreMesh(core_axis_name="c", subcore_axis_name="s")

    @functools.partial(
        pl.kernel,
        mesh=mesh,
        compiler_params=pltpu.CompilerParams(use_tc_tiling_on_sc=False),
        out_type=jax.ShapeDtypeStruct((B, D), jnp.bfloat16),
        scratch_types=[
            pltpu.VMEM((b_per_w,), jnp.int32),
            pltpu.VMEM((C, D), jnp.bfloat16),
            pltpu.VMEM((C, D), jnp.bfloat16),
            pltpu.SemaphoreType.DMA,
            pltpu.SemaphoreType.DMA,
            pltpu.SemaphoreType.DMA,
            pltpu.SemaphoreType.DMA,
        ],
    )
    def k(table_hbm, idx_hbm, out_hbm, idx_v, rows0, rows1, g0, g1, s0, s1):
        wid = lax.axis_index("s") * NC + lax.axis_index("c")
        base = wid * b_per_w
        rows = (rows0, rows1)
        gsem = (g0, g1)
        ssem = (s0, s1)

        pltpu.sync_copy(idx_hbm.at[pl.ds(base, b_per_w)], idx_v)

        def gather(g, b):
            return pltpu.make_async_copy(
                table_hbm.at[idx_v.at[pl.ds(g * C, C)]], rows[b], gsem[b]
            )

        def store(g, b):
            return pltpu.make_async_copy(
                rows[b], out_hbm.at[pl.ds(base + g * C, C)], ssem[b]
            )

        gather(0, 0).start()
        gather(1, 1).start()

        def body(p, carry):
            g = p * 2
            for b in (0, 1):
                gather(g + b, b).wait()
                store(g + b, b).start()
                store(g + b, b).wait()
                gather(g + b + 2, b).start()
            return carry

        lax.fori_loop(0, n_pairs - 1, body, 0)

        g = n_chunks - 2
        for b in (0, 1):
            gather(g + b, b).wait()
            store(g + b, b).start()
            store(g + b, b).wait()

    return k


def kernel(sequence, weight):
    Bs, H = sequence.shape
    V, D = weight.shape
    idx = sequence.reshape(-1).astype(jnp.int32)
    wb = weight.astype(jnp.bfloat16)
    out = _make_gather(V, D, Bs * H)(wb, idx)
    return out.reshape(Bs, H, D).astype(jnp.float32)
